# baseline (speedup 1.0000x reference)
import functools

import jax
import jax.numpy as jnp
from jax import lax
from jax.experimental import pallas as pl
from jax.experimental.pallas import tpu as pltpu
from jax.experimental.pallas import tpu_sc as plsc

B = 16384
D = 64
N_ROWS = 1000000
BC = 128
NB = 3907
S = BC * NB
NCB = 7813
N_CORES = 2
N_SUBCORES = 16
N_WORKERS = N_CORES * N_SUBCORES
BPW = B // N_WORKERS
LANES = 16
CH = 256
NPASS = BPW // CH
NBLK = CH // LANES


def _tp_body(in1_ref, in2_ref, out_ref):
    eye = jnp.eye(D, dtype=jnp.float32)
    dn = (((0,), (0,)), ((), ()))
    a = jax.lax.dot_general(in1_ref[...], eye, dn,
                            preferred_element_type=jnp.float32)
    b = jax.lax.dot_general(in2_ref[...], eye, dn,
                            preferred_element_type=jnp.float32)
    out_ref[:, 0:D] = a
    out_ref[:, D:2 * D] = b


def _pack(table_t):
    grid = (NB,)
    return pl.pallas_call(
        _tp_body,
        grid=grid,
        in_specs=[
            pl.BlockSpec((D, BC), lambda i: (0, i)),
            pl.BlockSpec((D, BC), lambda i: (0, jnp.minimum(NB + i, NCB - 1))),
        ],
        out_specs=pl.BlockSpec((BC, 2 * D), lambda i: (i, 0)),
        out_shape=jax.ShapeDtypeStruct((S, 2 * D), jnp.float32),
    )(table_t, table_t)


def _tec_body(h_hbm, t_hbm, r_hbm, ent_hbm, rel_hbm, dummy_hbm, out_hbm,
              hidx_v, tidx_v, ridx_v, he_v, te_v, re_v, out_v, sem):
    cid = lax.axis_index("c")
    sid = lax.axis_index("s")
    wid = sid * N_CORES + cid
    base = wid * BPW

    pltpu.sync_copy(h_hbm.at[pl.ds(base, BPW)], hidx_v)
    pltpu.sync_copy(t_hbm.at[pl.ds(base, BPW)], tidx_v)
    pltpu.sync_copy(r_hbm.at[pl.ds(base, BPW)], ridx_v)

    lane = jnp.arange(LANES, dtype=jnp.int32)
    nh = jnp.int32(S)

    def half(p, carry0):
        def fire(g, carry):
            off = p * CH + g * LANES
            hv = hidx_v[pl.ds(off, LANES)]
            tv = tidx_v[pl.ds(off, LANES)]
            rv = ridx_v[pl.ds(off, LANES)]
            for k in range(LANES):
                dst = pl.ds(g * LANES + k, 1)
                hk = hv[k]
                tk = tv[k]
                rk = rv[k]
                hk = jnp.where(hk >= nh, hk - nh, hk)
                tk = jnp.where(tk >= nh, tk - nh, tk)
                rk = jnp.where(rk >= nh, rk - nh, rk)
                pltpu.async_copy(ent_hbm.at[pl.ds(hk, 1), :],
                                 he_v.at[dst, :], sem)
                pltpu.async_copy(ent_hbm.at[pl.ds(tk, 1), :],
                                 te_v.at[dst, :], sem)
                pltpu.async_copy(rel_hbm.at[pl.ds(rk, 1), :],
                                 re_v.at[dst, :], sem)
            return carry

        lax.fori_loop(0, CH // LANES, fire, 0)
        pltpu.make_async_copy(dummy_hbm, he_v, sem).wait()
        pltpu.make_async_copy(dummy_hbm, te_v, sem).wait()
        pltpu.make_async_copy(dummy_hbm, re_v, sem).wait()

        def block(b, carry):
            rows = b * LANES + lane
            off = p * CH + b * LANES
            hv16 = hidx_v[pl.ds(off, LANES)]
            tv16 = tidx_v[pl.ds(off, LANES)]
            rv16 = ridx_v[pl.ds(off, LANES)]
            zero = jnp.zeros((LANES,), jnp.int32)
            hoff = jnp.where(hv16 >= nh, jnp.int32(D), zero)
            toff = jnp.where(tv16 >= nh, jnp.int32(D), zero)
            roff = jnp.where(rv16 >= nh, jnp.int32(D), zero)

            def dcol(d, acc):
                hv = plsc.load_gather(he_v, [rows, hoff + d])
                tv = plsc.load_gather(te_v, [rows, toff + d])
                rv = plsc.load_gather(re_v, [rows, roff + d])
                e = hv + rv - tv
                return acc + e * e

            s = lax.fori_loop(0, D, dcol, jnp.zeros((LANES,), jnp.float32))
            s = s + jnp.float32(1e-12)
            i = plsc.bitcast(s, jnp.int32)
            y = plsc.bitcast(jnp.int32(0x5F3759DF) - (i >> 1), jnp.float32)
            half_s = jnp.float32(0.5) * s
            for _ in range(3):
                y = y * (jnp.float32(1.5) - half_s * y * y)
            out_v[pl.ds(p * CH + b * LANES, LANES)] = -(s * y)
            return carry

        lax.fori_loop(0, NBLK, block, 0)
        return carry0

    lax.fori_loop(0, NPASS, half, 0)
    pltpu.sync_copy(out_v, out_hbm.at[pl.ds(base, BPW)])


@jax.jit
def _score(triples, entity_emb, relation_emb):
    ent_pack = _pack(entity_emb.T)
    rel_pack = _pack(relation_emb.T)

    mesh = plsc.VectorSubcoreMesh(core_axis_name="c", subcore_axis_name="s")
    run = functools.partial(
        pl.kernel,
        mesh=mesh,
        compiler_params=pltpu.CompilerParams(needs_layout_passes=False),
        out_type=jax.ShapeDtypeStruct((B,), jnp.float32),
        scratch_types=[
            pltpu.VMEM((BPW,), jnp.int32),
            pltpu.VMEM((BPW,), jnp.int32),
            pltpu.VMEM((BPW,), jnp.int32),
            pltpu.VMEM((CH, 2 * D), jnp.float32),
            pltpu.VMEM((CH, 2 * D), jnp.float32),
            pltpu.VMEM((CH, 2 * D), jnp.float32),
            pltpu.VMEM((BPW,), jnp.float32),
            pltpu.SemaphoreType.DMA,
        ],
    )(_tec_body)
    dummy = jnp.zeros((CH, 2 * D), jnp.float32)
    return run(triples[0], triples[1], triples[2], ent_pack, rel_pack, dummy)


def kernel(triples, entity_emb, relation_emb):
    return _score(triples.astype(jnp.int32), entity_emb, relation_emb)
